# baseline (speedup 1.0000x reference)
import functools
import jax
import jax.numpy as jnp
from jax import lax
from jax.experimental import pallas as pl
from jax.experimental.pallas import tpu as pltpu
from jax.experimental.pallas import tpu_sc as plsc

CAP = 100000
D = 256
VDIM = 64
K = 5

BLK = 16384
NBTC = 4
R_TC = NBTC * BLK
R_SC = CAP - R_TC

NW = 32
PER_W = 1080
CHUNK = 128
NCHUNK = 9
SC_PAD = 36864

G = BLK // 128
NEG = float("-inf")
IMAX = 2**31 - 1


def _top5_lanes(sim, idx, nlane):
    lane = lax.broadcasted_iota(jnp.int32, (1, nlane), 1)
    v_out = jnp.full((1, nlane), NEG, dtype=jnp.float32)
    i_out = jnp.zeros((1, nlane), dtype=jnp.int32)
    for j in range(K):
        m = jnp.max(sim)
        sel = jnp.min(jnp.where(sim == m, idx, IMAX))
        v_out = jnp.where(lane == j, m, v_out)
        i_out = jnp.where(lane == j, sel, i_out)
        sim = jnp.where((sim == m) & (idx == sel), NEG, sim)
    return v_out, i_out


def _qprep(query_ref, W1_ref, b1_ref, W2_ref, b2_ref, gamma_ref, beta_ref,
           qn_ref):
    q = query_ref[...]
    h = jnp.dot(q, W1_ref[...], preferred_element_type=jnp.float32) + b1_ref[...]
    h = h * jax.nn.sigmoid(h)
    h = jnp.dot(h, W2_ref[...], preferred_element_type=jnp.float32) + b2_ref[...]
    mean = jnp.mean(h, axis=-1, keepdims=True)
    var = jnp.mean((h - mean) * (h - mean), axis=-1, keepdims=True)
    h = (h - mean) * lax.rsqrt(var + 1e-5) * gamma_ref[...] + beta_ref[...]
    n = jnp.sqrt(jnp.sum(h * h))
    qn_ref[...] = h / jnp.maximum(n, 1e-12)


def _tc_stream(query_ref, W1_ref, b1_ref, W2_ref, b2_ref, gamma_ref,
               beta_ref, keys_ref, tv_ref, ti_ref, qn_ref):
    i = pl.program_id(0)

    @pl.when(i == 0)
    def _():
        q = query_ref[...]
        h = jnp.dot(q, W1_ref[...], preferred_element_type=jnp.float32) + b1_ref[...]
        h = h * jax.nn.sigmoid(h)
        h = jnp.dot(h, W2_ref[...], preferred_element_type=jnp.float32) + b2_ref[...]
        mean = jnp.mean(h, axis=-1, keepdims=True)
        var = jnp.mean((h - mean) * (h - mean), axis=-1, keepdims=True)
        h = (h - mean) * lax.rsqrt(var + 1e-5) * gamma_ref[...] + beta_ref[...]
        n = jnp.sqrt(jnp.sum(h * h))
        qn_ref[...] = h / jnp.maximum(n, 1e-12)

    x3 = keys_ref[...].reshape(G, 128, D)
    qn = qn_ref[...].reshape(1, 1, D)
    dot = jnp.sum(x3 * qn, axis=2)
    ss = jnp.sum(x3 * x3, axis=2)
    sim = dot / jnp.maximum(jnp.sqrt(ss), 1e-12)

    iota = (lax.broadcasted_iota(jnp.int32, (G, 128), 0) * 128
            + lax.broadcasted_iota(jnp.int32, (G, 128), 1) + i * BLK)
    lane = lax.broadcasted_iota(jnp.int32, (1, 1, 128), 2)
    vals_vec = jnp.full((1, 1, 128), NEG, dtype=jnp.float32)
    idx_vec = jnp.zeros((1, 1, 128), dtype=jnp.int32)
    for j in range(K):
        m = jnp.max(sim)
        sel = jnp.min(jnp.where(sim == m, iota, IMAX))
        vals_vec = jnp.where(lane == j, m, vals_vec)
        idx_vec = jnp.where(lane == j, sel, idx_vec)
        sim = jnp.where(iota == sel, NEG, sim)
    tv_ref[...] = vals_vec
    ti_ref[...] = idx_vec


def _sc_sims(qn_hbm, keys_hbm, dot_hbm, ss_hbm, qbuf, xbuf, dbuf, sbuf,
             tmp_d, tmp_s):
    c = lax.axis_index("c")
    s = lax.axis_index("s")
    wid = s * 2 + c
    wbase = jnp.minimum(wid * PER_W, R_SC - PER_W)
    pltpu.sync_copy(qn_hbm, qbuf)
    iota16 = lax.iota(jnp.int32, 16)

    @pl.loop(0, NCHUNK)
    def _chunk(ci):
        cbase = jnp.minimum(ci * CHUNK, PER_W - CHUNK)
        row0 = R_TC + wbase + cbase
        pltpu.sync_copy(keys_hbm.at[pl.ds(row0, CHUNK), :], xbuf)

        qv = [qbuf[pl.ds(16 * t, 16)] for t in range(16)]

        @pl.loop(0, CHUNK // 16)
        def _batch(b):
            @pl.loop(0, 16)
            def _row(k):
                acc_d = None
                acc_s = None
                for t in range(16):
                    xv = xbuf[b * 16 + k, pl.ds(16 * t, 16)]
                    pd = xv * qv[t]
                    ps = xv * xv
                    acc_d = pd if acc_d is None else acc_d + pd
                    acc_s = ps if acc_s is None else acc_s + ps
                tmp_d[pl.ds(k * 16, 16)] = plsc.cumsum(acc_d)
                tmp_s[pl.ds(k * 16, 16)] = plsc.cumsum(acc_s)

            idxv = iota16 * 16 + 15
            dbuf[pl.ds(b * 16, 16)] = plsc.load_gather(tmp_d, [idxv])
            sbuf[pl.ds(b * 16, 16)] = plsc.load_gather(tmp_s, [idxv])

        pltpu.sync_copy(dbuf, dot_hbm.at[pl.ds(wbase + cbase, CHUNK)])
        pltpu.sync_copy(sbuf, ss_hbm.at[pl.ds(wbase + cbase, CHUNK)])


def _merge(tv_ref, ti_ref, dot_ref, ss_ref, values_ref, out_ref,
           rows_ref, idx_smem, sem):
    SR = SC_PAD // 128
    flat = (lax.broadcasted_iota(jnp.int32, (SR, 128), 0) * 128
            + lax.broadcasted_iota(jnp.int32, (SR, 128), 1))
    ssim = dot_ref[...] / jnp.maximum(jnp.sqrt(ss_ref[...]), 1e-12)
    ssim = jnp.where(flat < R_SC, ssim, NEG)
    scv, sci = _top5_lanes(ssim, flat + R_TC, 128)

    allv = jnp.concatenate([tv_ref[...].reshape(NBTC, 128), scv], axis=0)
    alli = jnp.concatenate([ti_ref[...].reshape(NBTC, 128), sci], axis=0)
    fv, fi = _top5_lanes(allv, alli, 128)

    lane = lax.broadcasted_iota(jnp.int32, (1, 128), 1)
    svals = [jnp.sum(jnp.where(lane == j, fv, 0.0)) for j in range(K)]
    for j in range(K):
        idx_smem[j] = jnp.sum(jnp.where(lane == j, fi, 0))
    for j in range(K):
        pltpu.make_async_copy(
            values_ref.at[pl.ds(idx_smem[j], 1), :],
            rows_ref.at[pl.ds(j, 1), :],
            sem,
        ).start()
    m0 = svals[0]
    es = [jnp.exp(sv - m0) for sv in svals]
    denom = es[0] + es[1] + es[2] + es[3] + es[4]
    for j in range(K):
        pltpu.make_async_copy(
            values_ref.at[pl.ds(idx_smem[j], 1), :],
            rows_ref.at[pl.ds(j, 1), :],
            sem,
        ).wait()
    rows = rows_ref[...]
    acc = (es[0] / denom) * rows[0:1, :]
    for j in range(1, K):
        acc = acc + (es[j] / denom) * rows[j:j + 1, :]
    out_ref[...] = acc


@jax.jit
def kernel(query, keys, values, W1, b1, W2, b2, gamma, beta):
    b1r = b1.reshape(1, D)
    b2r = b2.reshape(1, D)
    gr = gamma.reshape(1, D)
    br = beta.reshape(1, D)

    qn = pl.pallas_call(
        _qprep,
        out_shape=jax.ShapeDtypeStruct((1, D), jnp.float32),
    )(query, W1, b1r, W2, b2r, gr, br)

    sc_dot, sc_ss = pl.kernel(
        _sc_sims,
        out_type=[
            jax.ShapeDtypeStruct((SC_PAD,), jnp.float32),
            jax.ShapeDtypeStruct((SC_PAD,), jnp.float32),
        ],
        mesh=plsc.VectorSubcoreMesh(core_axis_name="c", subcore_axis_name="s"),
        compiler_params=pltpu.CompilerParams(needs_layout_passes=False),
        scratch_types=[
            pltpu.VMEM((D,), jnp.float32),
            pltpu.VMEM((CHUNK, D), jnp.float32),
            pltpu.VMEM((CHUNK,), jnp.float32),
            pltpu.VMEM((CHUNK,), jnp.float32),
            pltpu.VMEM((D,), jnp.float32),
            pltpu.VMEM((D,), jnp.float32),
        ],
    )(qn.reshape(D), keys)

    tv, ti = pl.pallas_call(
        _tc_stream,
        grid=(NBTC,),
        in_specs=[
            pl.BlockSpec((1, D), lambda i: (0, 0)),
            pl.BlockSpec((D, D), lambda i: (0, 0)),
            pl.BlockSpec((1, D), lambda i: (0, 0)),
            pl.BlockSpec((D, D), lambda i: (0, 0)),
            pl.BlockSpec((1, D), lambda i: (0, 0)),
            pl.BlockSpec((1, D), lambda i: (0, 0)),
            pl.BlockSpec((1, D), lambda i: (0, 0)),
            pl.BlockSpec((BLK, D), lambda i: (i, 0)),
        ],
        out_specs=[
            pl.BlockSpec((1, 1, 128), lambda i: (i, 0, 0)),
            pl.BlockSpec((1, 1, 128), lambda i: (i, 0, 0)),
        ],
        out_shape=[
            jax.ShapeDtypeStruct((NBTC, 1, 128), jnp.float32),
            jax.ShapeDtypeStruct((NBTC, 1, 128), jnp.int32),
        ],
        scratch_shapes=[pltpu.VMEM((1, D), jnp.float32)],
    )(query, W1, b1r, W2, b2r, gr, br, keys)

    out = pl.pallas_call(
        _merge,
        in_specs=[
            pl.BlockSpec((NBTC, 1, 128), lambda: (0, 0, 0)),
            pl.BlockSpec((NBTC, 1, 128), lambda: (0, 0, 0)),
            pl.BlockSpec((SC_PAD // 128, 128), lambda: (0, 0)),
            pl.BlockSpec((SC_PAD // 128, 128), lambda: (0, 0)),
            pl.BlockSpec(memory_space=pl.ANY),
        ],
        out_specs=pl.BlockSpec((1, VDIM), lambda: (0, 0)),
        out_shape=jax.ShapeDtypeStruct((1, VDIM), jnp.float32),
        scratch_shapes=[
            pltpu.VMEM((8, VDIM), jnp.float32),
            pltpu.SMEM((8,), jnp.int32),
            pltpu.SemaphoreType.DMA,
        ],
    )(tv, ti, sc_dot.reshape(SC_PAD // 128, 128),
      sc_ss.reshape(SC_PAD // 128, 128), values)

    return out.reshape(VDIM)


# speedup vs baseline: 1.2392x; 1.2392x over previous
import functools
import jax
import jax.numpy as jnp
from jax.experimental import pallas as pl
from jax.experimental.pallas import tpu as pltpu

CAP = 100000
D = 256
VDIM = 64
K = 5
BLK = 16384
NB = (CAP + BLK - 1) // BLK
G = BLK // 128

NEG = float("-inf")
IMAX = 2**31 - 1


def _stream_kernel(query_ref, W1_ref, b1_ref, W2_ref, b2_ref, gamma_ref,
                   beta_ref, keys_ref, tv_ref, ti_ref, qn_ref):
    i = pl.program_id(0)

    @pl.when(i == 0)
    def _():
        q = query_ref[...]
        h = jnp.dot(q, W1_ref[...], preferred_element_type=jnp.float32) + b1_ref[...]
        h = h * jax.nn.sigmoid(h)
        h = jnp.dot(h, W2_ref[...], preferred_element_type=jnp.float32) + b2_ref[...]
        mean = jnp.mean(h, axis=-1, keepdims=True)
        var = jnp.mean((h - mean) * (h - mean), axis=-1, keepdims=True)
        h = (h - mean) * jax.lax.rsqrt(var + 1e-5) * gamma_ref[...] + beta_ref[...]
        n = jnp.sqrt(jnp.sum(h * h))
        qn_ref[...] = h / jnp.maximum(n, 1e-12)

    x3 = keys_ref[...].reshape(G, 128, D)
    qn = qn_ref[...].reshape(1, 1, D)
    dot = jnp.sum(x3 * qn, axis=2)
    ss = jnp.sum(x3 * x3, axis=2)
    sim = dot / jnp.maximum(jnp.sqrt(ss), 1e-12)

    iota = (jax.lax.broadcasted_iota(jnp.int32, (G, 128), 0) * 128
            + jax.lax.broadcasted_iota(jnp.int32, (G, 128), 1) + i * BLK)
    sim = jnp.where(iota < CAP, sim, NEG)
    lane = jax.lax.broadcasted_iota(jnp.int32, (1, 1, 128), 2)
    vals_vec = jnp.full((1, 1, 128), NEG, dtype=jnp.float32)
    idx_vec = jnp.zeros((1, 1, 128), dtype=jnp.int32)
    for j in range(K):
        m = jnp.max(sim)
        sel = jnp.min(jnp.where(sim == m, iota, IMAX))
        vals_vec = jnp.where(lane == j, m, vals_vec)
        idx_vec = jnp.where(lane == j, sel, idx_vec)
        sim = jnp.where(iota == sel, NEG, sim)
    tv_ref[...] = vals_vec
    ti_ref[...] = idx_vec


def _merge_kernel(tv_ref, ti_ref, w_ref, idx_ref):
    tv = tv_ref[...]
    ti = ti_ref[...]
    pos = (jax.lax.broadcasted_iota(jnp.int32, (NB, 1, 128), 0) * 128
           + jax.lax.broadcasted_iota(jnp.int32, (NB, 1, 128), 2))
    lane = jax.lax.broadcasted_iota(jnp.int32, (1, 128), 1)

    sims = []
    idxs = []
    for j in range(K):
        m = jnp.max(tv)
        p = jnp.min(jnp.where(tv == m, pos, IMAX))
        ridx = jnp.max(jnp.where(pos == p, ti, 0))
        sims.append(m)
        idxs.append(ridx)
        tv = jnp.where(pos == p, NEG, tv)

    m0 = sims[0]
    es = [jnp.exp(s - m0) for s in sims]
    denom = es[0] + es[1] + es[2] + es[3] + es[4]

    w_out = jnp.zeros((1, 128), dtype=jnp.float32)
    i_out = jnp.zeros((1, 128), dtype=jnp.int32)
    for j in range(K):
        w_out = jnp.where(lane == j, es[j] / denom, w_out)
        i_out = jnp.where(lane == j, idxs[j], i_out)
    w_ref[...] = w_out
    idx_ref[...] = i_out


def _gather_kernel(idx_ref, w_ref, values_ref, out_ref):
    i = pl.program_id(0)

    @pl.when(i == 0)
    def _():
        out_ref[...] = jnp.zeros_like(out_ref)

    out_ref[...] += w_ref[i] * values_ref[0]


@jax.jit
def kernel(query, keys, values, W1, b1, W2, b2, gamma, beta):
    b1r = b1.reshape(1, D)
    b2r = b2.reshape(1, D)
    gr = gamma.reshape(1, D)
    br = beta.reshape(1, D)

    tv, ti = pl.pallas_call(
        _stream_kernel,
        grid=(NB,),
        in_specs=[
            pl.BlockSpec((1, D), lambda i: (0, 0)),
            pl.BlockSpec((D, D), lambda i: (0, 0)),
            pl.BlockSpec((1, D), lambda i: (0, 0)),
            pl.BlockSpec((D, D), lambda i: (0, 0)),
            pl.BlockSpec((1, D), lambda i: (0, 0)),
            pl.BlockSpec((1, D), lambda i: (0, 0)),
            pl.BlockSpec((1, D), lambda i: (0, 0)),
            pl.BlockSpec((BLK, D), lambda i: (i, 0)),
        ],
        out_specs=[
            pl.BlockSpec((1, 1, 128), lambda i: (i, 0, 0)),
            pl.BlockSpec((1, 1, 128), lambda i: (i, 0, 0)),
        ],
        out_shape=[
            jax.ShapeDtypeStruct((NB, 1, 128), jnp.float32),
            jax.ShapeDtypeStruct((NB, 1, 128), jnp.int32),
        ],
        scratch_shapes=[pltpu.VMEM((1, D), jnp.float32)],
    )(query, W1, b1r, W2, b2r, gr, br, keys)

    w, idx = pl.pallas_call(
        _merge_kernel,
        out_shape=[
            jax.ShapeDtypeStruct((1, 128), jnp.float32),
            jax.ShapeDtypeStruct((1, 128), jnp.int32),
        ],
    )(tv, ti)

    values3 = values.reshape(CAP, 1, VDIM)
    out = pl.pallas_call(
        _gather_kernel,
        grid_spec=pltpu.PrefetchScalarGridSpec(
            num_scalar_prefetch=2,
            grid=(K,),
            in_specs=[
                pl.BlockSpec((1, 1, VDIM), lambda i, idx_ref, w_ref: (idx_ref[i], 0, 0)),
            ],
            out_specs=pl.BlockSpec((1, VDIM), lambda i, idx_ref, w_ref: (0, 0)),
        ),
        out_shape=jax.ShapeDtypeStruct((1, VDIM), jnp.float32),
    )(idx[0], w[0], values3)

    return out.reshape(VDIM)
